# baseline (speedup 1.0000x reference)
import functools

import jax
import jax.numpy as jnp
from jax import lax
from jax.experimental import pallas as pl
from jax.experimental.pallas import tpu as pltpu
from jax.experimental.pallas import tpu_sc as plsc

_N = 10000
_NPAD = 10240
_E = 320000
_NC = 2
_NS = 16
_NW = _NC * _NS
_EPW = _E // _NW
_UNROLL = 5
_GB = 8
_GN = _NW // _GB


def _leaky(x, s):
    return jnp.where(x > 0, x, s * x)


def _bn_leaky(z, g, b, slope):
    ones = jnp.ones((1, _N), jnp.float32)
    s1 = jnp.dot(ones, z, preferred_element_type=jnp.float32)
    s2 = jnp.dot(ones, z * z, preferred_element_type=jnp.float32)
    m = s1 * (1.0 / _N)
    v = s2 * (1.0 / _N) - m * m
    return _leaky((z - m) / jnp.sqrt(v + 1e-5) * g + b, slope)


_SC_CH = 4
_SB = _N // _SC_CH


def _scoring_body(emb_hbm, W1, b1, g1, be1, W2, b2, g2, be2, W3, b3, out,
                  bufs, sems):
    copies = [
        pltpu.make_async_copy(emb_hbm.at[pl.ds(c * _SB, _SB), :],
                              bufs.at[c], sems.at[c])
        for c in range(_SC_CH)
    ]
    for c in copies:
        c.start()

    ones = jnp.ones((1, _SB), jnp.float32)
    z1s = []
    s1 = None
    s2 = None
    for ci, c in enumerate(copies):
        c.wait()
        x = bufs[ci]
        z = jnp.dot(x, W1[...], preferred_element_type=jnp.float32) + b1[...]
        z1s.append(z)
        p1 = jnp.dot(ones, z, preferred_element_type=jnp.float32)
        p2 = jnp.dot(ones, z * z, preferred_element_type=jnp.float32)
        s1 = p1 if s1 is None else s1 + p1
        s2 = p2 if s2 is None else s2 + p2

    m = s1 * (1.0 / _N)
    v = s2 * (1.0 / _N) - m * m
    rstd = 1.0 / jnp.sqrt(v + 1e-5)

    z2s = []
    s1 = None
    s2 = None
    for z in z1s:
        h = _leaky((z - m) * rstd * g1[...] + be1[...], 0.2)
        z2 = jnp.dot(h, W2[...], preferred_element_type=jnp.float32) + b2[...]
        z2s.append(z2)
        p1 = jnp.dot(ones, z2, preferred_element_type=jnp.float32)
        p2 = jnp.dot(ones, z2 * z2, preferred_element_type=jnp.float32)
        s1 = p1 if s1 is None else s1 + p1
        s2 = p2 if s2 is None else s2 + p2

    m = s1 * (1.0 / _N)
    v = s2 * (1.0 / _N) - m * m
    rstd = 1.0 / jnp.sqrt(v + 1e-5)

    for ci, z2 in enumerate(z2s):
        h = _leaky((z2 - m) * rstd * g2[...] + be2[...], 0.2)
        s = jnp.dot(h, W3[...], preferred_element_type=jnp.float32) + b3[...]
        srow = s.T
        esrow = jnp.exp(srow)
        o = ci * _SB
        out[:, o:o + _SB] = srow
        out[:, _NPAD + o:_NPAD + o + _SB] = esrow
        out[:, 2 * _NPAD + o:2 * _NPAD + o + _SB] = esrow * srow

    zpad = jnp.zeros((1, _NPAD - _N), jnp.float32)
    out[:, _N:_NPAD] = zpad
    out[:, _NPAD + _N:2 * _NPAD] = zpad
    out[:, 2 * _NPAD + _N:3 * _NPAD] = zpad


def _scoring(emb, W1, b1, g1, be1, W2, b2, g2, be2, W3, b3):
    full = lambda s: pl.BlockSpec(s, lambda: tuple(0 for _ in s))
    return pl.pallas_call(
        _scoring_body,
        in_specs=[pl.BlockSpec(memory_space=pl.ANY)] + [
            full((128, 64)), full((64,)), full((64,)), full((64,)),
            full((64, 32)), full((32,)), full((32,)), full((32,)),
            full((32, 1)), full((1,)),
        ],
        out_specs=full((1, 3 * _NPAD)),
        out_shape=jax.ShapeDtypeStruct((1, 3 * _NPAD), jnp.float32),
        scratch_shapes=[
            pltpu.VMEM((_SC_CH, _SB, 128), jnp.float32),
            pltpu.SemaphoreType.DMA((_SC_CH,)),
        ],
    )(emb, W1, b1, g1, be1, W2, b2, g2, be2, W3, b3)


def _edge_body(with_deg, tab_hbm, src_hbm, dst_hbm, w_hbm, out_hbm,
               tab_v, src_v, dst_v, w_v, sems, *accs):
    cid = lax.axis_index("c")
    sid = lax.axis_index("s")
    wid = sid * _NC + cid
    base = wid * _EPW

    c0 = pltpu.make_async_copy(tab_hbm.at[pl.ds(_NPAD, 2 * _NPAD)], tab_v,
                               sems.at[0])
    c1 = pltpu.make_async_copy(src_hbm.at[pl.ds(base, _EPW)], src_v,
                               sems.at[1])
    c2 = pltpu.make_async_copy(dst_hbm.at[pl.ds(base, _EPW)], dst_v,
                               sems.at[2])
    c3 = pltpu.make_async_copy(w_hbm.at[pl.ds(base, _EPW)], w_v, sems.at[3])
    c0.start()
    c1.start()
    c2.start()
    c3.start()

    zero16 = jnp.zeros((16,), jnp.float32)

    def zbody(i, _):
        o = i * 64
        for j in range(4):
            for a in accs:
                a[pl.ds(o + j * 16, 16)] = zero16
        return 0

    lax.fori_loop(0, _NPAD // 64, zbody, 0)

    c0.wait()
    c1.wait()
    c2.wait()
    c3.wait()

    ones16 = jnp.ones((16,), jnp.float32)

    @plsc.parallel_loop(0, _EPW // 16, step=1, unroll=_UNROLL)
    def _eloop(i):
        o = i * 16
        isrc = src_v[pl.ds(o, 16)]
        idst = dst_v[pl.ds(o, 16)]
        wv = w_v[pl.ds(o, 16)]
        es_s = plsc.load_gather(tab_v, [isrc])
        t_s = plsc.load_gather(tab_v, [isrc + _NPAD])
        es_d = plsc.load_gather(tab_v, [idst])
        t_d = plsc.load_gather(tab_v, [idst + _NPAD])
        plsc.addupdate_scatter(accs[0], [idst], es_s)
        plsc.addupdate_scatter(accs[1], [idst], wv * t_s)
        plsc.addupdate_scatter(accs[2], [isrc], es_d)
        plsc.addupdate_scatter(accs[3], [isrc], wv * t_d)
        if with_deg:
            plsc.addupdate_scatter(accs[4], [idst], ones16)

    nq = len(accs)
    outcopies = [
        pltpu.make_async_copy(
            a, out_hbm.at[pl.ds((wid * nq + q) * _NPAD, _NPAD)], sems.at[q])
        for q, a in enumerate(accs)
    ]
    for c in outcopies:
        c.start()
    for c in outcopies:
        c.wait()


@functools.lru_cache(maxsize=None)
def _make_edge(with_deg):
    nq = 5 if with_deg else 4
    scratch = [
        pltpu.VMEM((2 * _NPAD,), jnp.float32),
        pltpu.VMEM((_EPW,), jnp.int32),
        pltpu.VMEM((_EPW,), jnp.int32),
        pltpu.VMEM((_EPW,), jnp.float32),
        pltpu.SemaphoreType.DMA((5,)),
    ] + [pltpu.VMEM((_NPAD,), jnp.float32) for _ in range(nq)]
    return pl.kernel(
        functools.partial(_edge_body, with_deg),
        out_type=jax.ShapeDtypeStruct((_NW * nq * _NPAD,), jnp.float32),
        mesh=plsc.VectorSubcoreMesh(core_axis_name="c", subcore_axis_name="s",
                                    num_cores=_NC, num_subcores=_NS),
        scratch_types=scratch,
        compiler_params=pltpu.CompilerParams(needs_layout_passes=False),
    )


def _edge_deg(*args):
    return _make_edge(True)(*args)


def _edge_nodeg(*args):
    return _make_edge(False)(*args)


def _node_body(alpha, final, parts_ref, s3_ref, orig3_ref,
               U1_ref, ub1_ref, ug_ref, ube_ref, U2_ref, ub2_ref, U3_ref,
               ub3_ref, *rest):
    nq = 4 if final else 5
    if final:
        scal_ref, deg_ref, out_ref, acc_ref = rest
    else:
        out_ref, deg_out_ref, acc_ref = rest

    g = pl.program_id(0)
    chunk = parts_ref[0:nq * _NPAD]
    for k in range(1, _GB):
        chunk = chunk + parts_ref[k * nq * _NPAD:(k + 1) * nq * _NPAD]

    @pl.when(g == 0)
    def _():
        acc_ref[...] = chunk

    @pl.when(g > 0)
    def _():
        acc_ref[...] = acc_ref[...] + chunk

    @pl.when(g == _GN - 1)
    def _():
        acc = acc_ref[...]

        def plane(q):
            return acc[q * _NPAD:(q + 1) * _NPAD].reshape(1, _NPAD)

        den_in = plane(0)
        num_in = plane(1)
        den_out = plane(2)
        num_out = plane(3)
        s_in = num_in / jnp.maximum(den_in, 1e-20)
        s_out = num_out / jnp.maximum(den_out, 1e-20)
        s = s3_ref[:, 0:_NPAD]

        mask = (lax.broadcasted_iota(jnp.int32, (1, _NPAD), 1) < _N).astype(
            jnp.float32)

        U1T = U1_ref[...].T
        ub1 = ub1_ref[...].reshape(-1, 1)
        ug = ug_ref[...].reshape(-1, 1)
        ube = ube_ref[...].reshape(-1, 1)
        U2T = U2_ref[...].T
        ub2 = ub2_ref[...].reshape(-1, 1)
        U3T = U3_ref[...].T
        ub3 = ub3_ref[...].reshape(1, 1)

        u1 = (U1T[:, 0:1] * s + U1T[:, 1:2] * s_in
              + U1T[:, 2:3] * s_out + ub1)
        u1 = u1 * mask
        m = jnp.sum(u1, axis=1, keepdims=True) * (1.0 / _N)
        v = jnp.sum(u1 * u1, axis=1, keepdims=True) * (1.0 / _N) - m * m
        h = _leaky((u1 - m) / jnp.sqrt(v + 1e-5) * ug + ube, 0.1)
        u2 = _leaky(
            jnp.dot(U2T, h, preferred_element_type=jnp.float32) + ub2, 0.1)
        u3 = jax.nn.sigmoid(
            jnp.dot(U3T, u2, preferred_element_type=jnp.float32) + ub3)
        orig = orig3_ref[:, 0:_NPAD]
        sn = alpha * u3 + (1.0 - alpha) * orig

        if not final:
            es = jnp.exp(sn)
            out_ref[:, 0:_NPAD] = sn
            out_ref[:, _NPAD:2 * _NPAD] = es
            out_ref[:, 2 * _NPAD:3 * _NPAD] = es * sn
            deg_out_ref[...] = plane(4)
        else:
            scal = scal_ref[...]
            gamma_p = scal[0, 0]
            beta_p = scal[0, 1]
            cscale = scal[0, 2]
            mixraw = scal[0, 3]
            deg = deg_ref[...]
            c_v = jnp.log(deg * cscale + 1e-6)
            c_smooth = 5.0 * jnp.tanh((gamma_p * c_v + beta_p) * 0.2)
            mix = jax.nn.sigmoid(mixraw)
            out_ref[...] = (mix * (sn * jax.nn.sigmoid(c_smooth))
                            + (1.0 - mix) * sn)


def _full2d(shape):
    return pl.BlockSpec(shape, lambda w: (0, 0))


def _node_first(parts, s3, orig3, U1l, ub1l, ugl, ubel, U2l, ub2l, U3l,
                ub3l):
    nq = 5
    return pl.pallas_call(
        functools.partial(_node_body, 0.8, False),
        grid=(_GN,),
        in_specs=[
            pl.BlockSpec((_GB * nq * _NPAD,), lambda w: (w,)),
            _full2d((1, 3 * _NPAD)),
            _full2d((1, 3 * _NPAD)),
            pl.BlockSpec((3, 24), lambda w: (0, 0)),
            pl.BlockSpec((24,), lambda w: (0,)),
            pl.BlockSpec((24,), lambda w: (0,)),
            pl.BlockSpec((24,), lambda w: (0,)),
            pl.BlockSpec((24, 12), lambda w: (0, 0)),
            pl.BlockSpec((12,), lambda w: (0,)),
            pl.BlockSpec((12, 1), lambda w: (0, 0)),
            pl.BlockSpec((1,), lambda w: (0,)),
        ],
        out_specs=(_full2d((1, 3 * _NPAD)), _full2d((1, _NPAD))),
        out_shape=(
            jax.ShapeDtypeStruct((1, 3 * _NPAD), jnp.float32),
            jax.ShapeDtypeStruct((1, _NPAD), jnp.float32),
        ),
        scratch_shapes=[pltpu.VMEM((nq * _NPAD,), jnp.float32)],
    )(parts, s3, orig3, U1l, ub1l, ugl, ubel, U2l, ub2l, U3l, ub3l)


def _node_final(parts, s3, orig3, U1l, ub1l, ugl, ubel, U2l, ub2l, U3l,
                ub3l, scal, deg_row):
    nq = 4
    return pl.pallas_call(
        functools.partial(_node_body, 0.9, True),
        grid=(_GN,),
        in_specs=[
            pl.BlockSpec((_GB * nq * _NPAD,), lambda w: (w,)),
            _full2d((1, 3 * _NPAD)),
            _full2d((1, 3 * _NPAD)),
            pl.BlockSpec((3, 24), lambda w: (0, 0)),
            pl.BlockSpec((24,), lambda w: (0,)),
            pl.BlockSpec((24,), lambda w: (0,)),
            pl.BlockSpec((24,), lambda w: (0,)),
            pl.BlockSpec((24, 12), lambda w: (0, 0)),
            pl.BlockSpec((12,), lambda w: (0,)),
            pl.BlockSpec((12, 1), lambda w: (0, 0)),
            pl.BlockSpec((1,), lambda w: (0,)),
            _full2d((1, 4)),
            _full2d((1, _NPAD)),
        ],
        out_specs=_full2d((1, _NPAD)),
        out_shape=jax.ShapeDtypeStruct((1, _NPAD), jnp.float32),
        scratch_shapes=[pltpu.VMEM((nq * _NPAD,), jnp.float32)],
    )(parts, s3, orig3, U1l, ub1l, ugl, ubel, U2l, ub2l, U3l, ub3l,
      scal, deg_row)


def kernel(embeddings, edge_index, edge_weight, W1, b1, g1, be1, W2, b2, g2,
           be2, W3, b3, U1, ub1, ug, ube, U2, ub2, U3, ub3, gamma_p, beta_p,
           cscale, mixraw):
    src = edge_index[0]
    dst = edge_index[1]

    s3 = _scoring(embeddings, W1, b1, g1, be1, W2, b2, g2, be2, W3, b3)

    scal = jnp.stack([gamma_p, beta_p, cscale, mixraw]).reshape(1, 4)

    def layer_args(l):
        return (U1[l], ub1[l], ug[l], ube[l], U2[l], ub2[l], U3[l], ub3[l])

    parts0 = _edge_deg(s3.reshape(3 * _NPAD), src, dst, edge_weight)
    s3n, deg_row = _node_first(parts0, s3, s3, *layer_args(0))

    parts1 = _edge_nodeg(s3n.reshape(3 * _NPAD), src, dst, edge_weight)
    fin = _node_final(parts1, s3n, s3, *layer_args(1), scal, deg_row)

    return fin.reshape(_NPAD)[:_N]


# speedup vs baseline: 1.0485x; 1.0485x over previous
import functools

import jax
import jax.numpy as jnp
from jax import lax
from jax.experimental import pallas as pl
from jax.experimental.pallas import tpu as pltpu
from jax.experimental.pallas import tpu_sc as plsc

_N = 10000
_NPAD = 10240
_E = 320000
_NC = 2
_NS = 16
_NW = _NC * _NS
_EPW = _E // _NW
_UNROLL = 5
_GB = 16
_GN = _NW // _GB


def _leaky(x, s):
    return jnp.where(x > 0, x, s * x)


def _bn_leaky(z, g, b, slope):
    ones = jnp.ones((1, _N), jnp.float32)
    s1 = jnp.dot(ones, z, preferred_element_type=jnp.float32)
    s2 = jnp.dot(ones, z * z, preferred_element_type=jnp.float32)
    m = s1 * (1.0 / _N)
    v = s2 * (1.0 / _N) - m * m
    return _leaky((z - m) / jnp.sqrt(v + 1e-5) * g + b, slope)


def _scoring_body(emb, W1, b1, g1, be1, W2, b2, g2, be2, W3, b3, out):
    x = emb[...]
    z = jnp.dot(x, W1[...], preferred_element_type=jnp.float32) + b1[...]
    h = _bn_leaky(z, g1[...], be1[...], 0.2)
    z = jnp.dot(h, W2[...], preferred_element_type=jnp.float32) + b2[...]
    h = _bn_leaky(z, g2[...], be2[...], 0.2)
    s = jnp.dot(h, W3[...], preferred_element_type=jnp.float32) + b3[...]
    srow = s.T
    esrow = jnp.exp(srow)
    zpad = jnp.zeros((1, _NPAD - _N), jnp.float32)
    out[:, 0:_N] = srow
    out[:, _N:_NPAD] = zpad
    out[:, _NPAD:_NPAD + _N] = esrow
    out[:, _NPAD + _N:2 * _NPAD] = zpad
    out[:, 2 * _NPAD:2 * _NPAD + _N] = esrow * srow
    out[:, 2 * _NPAD + _N:3 * _NPAD] = zpad


def _scoring(emb, W1, b1, g1, be1, W2, b2, g2, be2, W3, b3):
    return pl.pallas_call(
        _scoring_body,
        out_shape=jax.ShapeDtypeStruct((1, 3 * _NPAD), jnp.float32),
    )(emb, W1, b1, g1, be1, W2, b2, g2, be2, W3, b3)


def _edge_body(with_deg, tab_hbm, src_hbm, dst_hbm, w_hbm, out_hbm,
               tab_v, src_v, dst_v, w_v, sems, *accs):
    cid = lax.axis_index("c")
    sid = lax.axis_index("s")
    wid = sid * _NC + cid
    base = wid * _EPW

    c0 = pltpu.make_async_copy(tab_hbm.at[pl.ds(_NPAD, 2 * _NPAD)], tab_v,
                               sems.at[0])
    c1 = pltpu.make_async_copy(src_hbm.at[pl.ds(base, _EPW)], src_v,
                               sems.at[1])
    c2 = pltpu.make_async_copy(dst_hbm.at[pl.ds(base, _EPW)], dst_v,
                               sems.at[2])
    c3 = pltpu.make_async_copy(w_hbm.at[pl.ds(base, _EPW)], w_v, sems.at[3])
    c0.start()
    c1.start()
    c2.start()
    c3.start()

    zero16 = jnp.zeros((16,), jnp.float32)

    def zbody(i, _):
        o = i * 64
        for j in range(4):
            for a in accs:
                a[pl.ds(o + j * 16, 16)] = zero16
        return 0

    lax.fori_loop(0, _NPAD // 64, zbody, 0)

    c0.wait()
    c1.wait()
    c2.wait()
    c3.wait()

    ones16 = jnp.ones((16,), jnp.float32)

    @plsc.parallel_loop(0, _EPW // 16, step=1, unroll=_UNROLL)
    def _eloop(i):
        o = i * 16
        isrc = src_v[pl.ds(o, 16)]
        idst = dst_v[pl.ds(o, 16)]
        wv = w_v[pl.ds(o, 16)]
        es_s = plsc.load_gather(tab_v, [isrc])
        t_s = plsc.load_gather(tab_v, [isrc + _NPAD])
        es_d = plsc.load_gather(tab_v, [idst])
        t_d = plsc.load_gather(tab_v, [idst + _NPAD])
        plsc.addupdate_scatter(accs[0], [idst], es_s)
        plsc.addupdate_scatter(accs[1], [idst], wv * t_s)
        plsc.addupdate_scatter(accs[2], [isrc], es_d)
        plsc.addupdate_scatter(accs[3], [isrc], wv * t_d)
        if with_deg:
            plsc.addupdate_scatter(accs[4], [idst], ones16)

    nq = len(accs)
    outcopies = [
        pltpu.make_async_copy(
            a, out_hbm.at[pl.ds((wid * nq + q) * _NPAD, _NPAD)], sems.at[q])
        for q, a in enumerate(accs)
    ]
    for c in outcopies:
        c.start()
    for c in outcopies:
        c.wait()


@functools.lru_cache(maxsize=None)
def _make_edge(with_deg):
    nq = 5 if with_deg else 4
    scratch = [
        pltpu.VMEM((2 * _NPAD,), jnp.float32),
        pltpu.VMEM((_EPW,), jnp.int32),
        pltpu.VMEM((_EPW,), jnp.int32),
        pltpu.VMEM((_EPW,), jnp.float32),
        pltpu.SemaphoreType.DMA((5,)),
    ] + [pltpu.VMEM((_NPAD,), jnp.float32) for _ in range(nq)]
    return pl.kernel(
        functools.partial(_edge_body, with_deg),
        out_type=jax.ShapeDtypeStruct((_NW * nq * _NPAD,), jnp.float32),
        mesh=plsc.VectorSubcoreMesh(core_axis_name="c", subcore_axis_name="s",
                                    num_cores=_NC, num_subcores=_NS),
        scratch_types=scratch,
        compiler_params=pltpu.CompilerParams(needs_layout_passes=False),
    )


def _edge_deg(*args):
    return _make_edge(True)(*args)


def _edge_nodeg(*args):
    return _make_edge(False)(*args)


def _node_body(alpha, final, parts_ref, s3_ref, orig3_ref,
               U1_ref, ub1_ref, ug_ref, ube_ref, U2_ref, ub2_ref, U3_ref,
               ub3_ref, *rest):
    nq = 4 if final else 5
    if final:
        scal_ref, deg_ref, out_ref, acc_ref = rest
    else:
        out_ref, deg_out_ref, acc_ref = rest

    g = pl.program_id(0)
    chunk = parts_ref[0:nq * _NPAD]
    for k in range(1, _GB):
        chunk = chunk + parts_ref[k * nq * _NPAD:(k + 1) * nq * _NPAD]

    @pl.when(g == 0)
    def _():
        acc_ref[...] = chunk

    @pl.when(g > 0)
    def _():
        acc_ref[...] = acc_ref[...] + chunk

    @pl.when(g == _GN - 1)
    def _():
        acc = acc_ref[...]

        def plane(q):
            return acc[q * _NPAD:(q + 1) * _NPAD].reshape(1, _NPAD)

        den_in = plane(0)
        num_in = plane(1)
        den_out = plane(2)
        num_out = plane(3)
        s_in = num_in / jnp.maximum(den_in, 1e-20)
        s_out = num_out / jnp.maximum(den_out, 1e-20)
        s = s3_ref[:, 0:_NPAD]

        mask = (lax.broadcasted_iota(jnp.int32, (1, _NPAD), 1) < _N).astype(
            jnp.float32)

        U1T = U1_ref[...].T
        ub1 = ub1_ref[...].reshape(-1, 1)
        ug = ug_ref[...].reshape(-1, 1)
        ube = ube_ref[...].reshape(-1, 1)
        U2T = U2_ref[...].T
        ub2 = ub2_ref[...].reshape(-1, 1)
        U3T = U3_ref[...].T
        ub3 = ub3_ref[...].reshape(1, 1)

        u1 = (U1T[:, 0:1] * s + U1T[:, 1:2] * s_in
              + U1T[:, 2:3] * s_out + ub1)
        u1 = u1 * mask
        m = jnp.sum(u1, axis=1, keepdims=True) * (1.0 / _N)
        v = jnp.sum(u1 * u1, axis=1, keepdims=True) * (1.0 / _N) - m * m
        h = _leaky((u1 - m) / jnp.sqrt(v + 1e-5) * ug + ube, 0.1)
        u2 = _leaky(
            jnp.dot(U2T, h, preferred_element_type=jnp.float32) + ub2, 0.1)
        u3 = jax.nn.sigmoid(
            jnp.dot(U3T, u2, preferred_element_type=jnp.float32) + ub3)
        orig = orig3_ref[:, 0:_NPAD]
        sn = alpha * u3 + (1.0 - alpha) * orig

        if not final:
            es = jnp.exp(sn)
            out_ref[:, 0:_NPAD] = sn
            out_ref[:, _NPAD:2 * _NPAD] = es
            out_ref[:, 2 * _NPAD:3 * _NPAD] = es * sn
            deg_out_ref[...] = plane(4)
        else:
            scal = scal_ref[...]
            gamma_p = scal[0, 0]
            beta_p = scal[0, 1]
            cscale = scal[0, 2]
            mixraw = scal[0, 3]
            deg = deg_ref[...]
            c_v = jnp.log(deg * cscale + 1e-6)
            c_smooth = 5.0 * jnp.tanh((gamma_p * c_v + beta_p) * 0.2)
            mix = jax.nn.sigmoid(mixraw)
            out_ref[...] = (mix * (sn * jax.nn.sigmoid(c_smooth))
                            + (1.0 - mix) * sn)


def _full2d(shape):
    return pl.BlockSpec(shape, lambda w: (0, 0))


def _node_first(parts, s3, orig3, U1l, ub1l, ugl, ubel, U2l, ub2l, U3l,
                ub3l):
    nq = 5
    return pl.pallas_call(
        functools.partial(_node_body, 0.8, False),
        grid=(_GN,),
        in_specs=[
            pl.BlockSpec((_GB * nq * _NPAD,), lambda w: (w,)),
            _full2d((1, 3 * _NPAD)),
            _full2d((1, 3 * _NPAD)),
            pl.BlockSpec((3, 24), lambda w: (0, 0)),
            pl.BlockSpec((24,), lambda w: (0,)),
            pl.BlockSpec((24,), lambda w: (0,)),
            pl.BlockSpec((24,), lambda w: (0,)),
            pl.BlockSpec((24, 12), lambda w: (0, 0)),
            pl.BlockSpec((12,), lambda w: (0,)),
            pl.BlockSpec((12, 1), lambda w: (0, 0)),
            pl.BlockSpec((1,), lambda w: (0,)),
        ],
        out_specs=(_full2d((1, 3 * _NPAD)), _full2d((1, _NPAD))),
        out_shape=(
            jax.ShapeDtypeStruct((1, 3 * _NPAD), jnp.float32),
            jax.ShapeDtypeStruct((1, _NPAD), jnp.float32),
        ),
        scratch_shapes=[pltpu.VMEM((nq * _NPAD,), jnp.float32)],
    )(parts, s3, orig3, U1l, ub1l, ugl, ubel, U2l, ub2l, U3l, ub3l)


def _node_final(parts, s3, orig3, U1l, ub1l, ugl, ubel, U2l, ub2l, U3l,
                ub3l, scal, deg_row):
    nq = 4
    return pl.pallas_call(
        functools.partial(_node_body, 0.9, True),
        grid=(_GN,),
        in_specs=[
            pl.BlockSpec((_GB * nq * _NPAD,), lambda w: (w,)),
            _full2d((1, 3 * _NPAD)),
            _full2d((1, 3 * _NPAD)),
            pl.BlockSpec((3, 24), lambda w: (0, 0)),
            pl.BlockSpec((24,), lambda w: (0,)),
            pl.BlockSpec((24,), lambda w: (0,)),
            pl.BlockSpec((24,), lambda w: (0,)),
            pl.BlockSpec((24, 12), lambda w: (0, 0)),
            pl.BlockSpec((12,), lambda w: (0,)),
            pl.BlockSpec((12, 1), lambda w: (0, 0)),
            pl.BlockSpec((1,), lambda w: (0,)),
            _full2d((1, 4)),
            _full2d((1, _NPAD)),
        ],
        out_specs=_full2d((1, _NPAD)),
        out_shape=jax.ShapeDtypeStruct((1, _NPAD), jnp.float32),
        scratch_shapes=[pltpu.VMEM((nq * _NPAD,), jnp.float32)],
    )(parts, s3, orig3, U1l, ub1l, ugl, ubel, U2l, ub2l, U3l, ub3l,
      scal, deg_row)


def kernel(embeddings, edge_index, edge_weight, W1, b1, g1, be1, W2, b2, g2,
           be2, W3, b3, U1, ub1, ug, ube, U2, ub2, U3, ub3, gamma_p, beta_p,
           cscale, mixraw):
    src = edge_index[0]
    dst = edge_index[1]

    s3 = _scoring(embeddings, W1, b1, g1, be1, W2, b2, g2, be2, W3, b3)

    scal = jnp.stack([gamma_p, beta_p, cscale, mixraw]).reshape(1, 4)

    def layer_args(l):
        return (U1[l], ub1[l], ug[l], ube[l], U2[l], ub2[l], U3[l], ub3[l])

    parts0 = _edge_deg(s3.reshape(3 * _NPAD), src, dst, edge_weight)
    s3n, deg_row = _node_first(parts0, s3, s3, *layer_args(0))

    parts1 = _edge_nodeg(s3n.reshape(3 * _NPAD), src, dst, edge_weight)
    fin = _node_final(parts1, s3n, s3, *layer_args(1), scal, deg_row)

    return fin.reshape(_NPAD)[:_N]


# speedup vs baseline: 1.0506x; 1.0020x over previous
import functools

import jax
import jax.numpy as jnp
from jax import lax
from jax.experimental import pallas as pl
from jax.experimental.pallas import tpu as pltpu
from jax.experimental.pallas import tpu_sc as plsc

_N = 10000
_NPAD = 10240
_E = 320000
_NC = 2
_NS = 16
_NW = _NC * _NS
_EPW = _E // _NW
_UNROLL = 5
_GB = 16
_GN = _NW // _GB


def _leaky(x, s):
    return jnp.where(x > 0, x, s * x)


def _bn_leaky(z, g, b, slope):
    ones = jnp.ones((1, _N), jnp.float32)
    s1 = jnp.dot(ones, z, preferred_element_type=jnp.float32)
    s2 = jnp.dot(ones, z * z, preferred_element_type=jnp.float32)
    m = s1 * (1.0 / _N)
    v = s2 * (1.0 / _N) - m * m
    return _leaky((z - m) / jnp.sqrt(v + 1e-5) * g + b, slope)


def _scoring_body(emb, W1, b1, g1, be1, W2, b2, g2, be2, W3, b3, out):
    x = emb[...]
    z = jnp.dot(x, W1[...], preferred_element_type=jnp.float32) + b1[...]
    h = _bn_leaky(z, g1[...], be1[...], 0.2)
    z = jnp.dot(h, W2[...], preferred_element_type=jnp.float32) + b2[...]
    h = _bn_leaky(z, g2[...], be2[...], 0.2)
    s = jnp.dot(h, W3[...], preferred_element_type=jnp.float32) + b3[...]
    srow = s.T
    esrow = jnp.exp(srow)
    zpad = jnp.zeros((1, _NPAD - _N), jnp.float32)
    out[:, 0:_N] = srow
    out[:, _N:_NPAD] = zpad
    out[:, _NPAD:_NPAD + _N] = esrow
    out[:, _NPAD + _N:2 * _NPAD] = zpad
    out[:, 2 * _NPAD:2 * _NPAD + _N] = esrow * srow
    out[:, 2 * _NPAD + _N:3 * _NPAD] = zpad


def _scoring(emb, W1, b1, g1, be1, W2, b2, g2, be2, W3, b3):
    return pl.pallas_call(
        _scoring_body,
        out_shape=jax.ShapeDtypeStruct((1, 3 * _NPAD), jnp.float32),
    )(emb, W1, b1, g1, be1, W2, b2, g2, be2, W3, b3)


def _edge_body(with_deg, tab_hbm, src_hbm, dst_hbm, w_hbm, out_hbm,
               tab_v, src_v, dst_v, w_v, sems, *accs):
    cid = lax.axis_index("c")
    sid = lax.axis_index("s")
    wid = sid * _NC + cid
    base = wid * _EPW

    c0 = pltpu.make_async_copy(tab_hbm.at[pl.ds(_NPAD, 2 * _NPAD)], tab_v,
                               sems.at[0])
    c1 = pltpu.make_async_copy(src_hbm.at[pl.ds(base, _EPW)], src_v,
                               sems.at[1])
    c2 = pltpu.make_async_copy(dst_hbm.at[pl.ds(base, _EPW)], dst_v,
                               sems.at[2])
    c3 = pltpu.make_async_copy(w_hbm.at[pl.ds(base, _EPW)], w_v, sems.at[3])
    c0.start()
    c1.start()
    c2.start()
    c3.start()

    zero16 = jnp.zeros((16,), jnp.float32)

    def zbody(i, _):
        o = i * 64
        for j in range(4):
            for a in accs:
                a[pl.ds(o + j * 16, 16)] = zero16
        return 0

    lax.fori_loop(0, _NPAD // 64, zbody, 0)

    c0.wait()
    c1.wait()
    c2.wait()
    c3.wait()

    ones16 = jnp.ones((16,), jnp.float32)

    @plsc.parallel_loop(0, _EPW // 16, step=1, unroll=_UNROLL)
    def _eloop(i):
        o = i * 16
        isrc = src_v[pl.ds(o, 16)]
        idst = dst_v[pl.ds(o, 16)]
        wv = w_v[pl.ds(o, 16)]
        es_s = plsc.load_gather(tab_v, [isrc])
        t_s = plsc.load_gather(tab_v, [isrc + _NPAD])
        es_d = plsc.load_gather(tab_v, [idst])
        t_d = plsc.load_gather(tab_v, [idst + _NPAD])
        plsc.addupdate_scatter(accs[0], [idst], es_s)
        plsc.addupdate_scatter(accs[1], [idst], wv * t_s)
        plsc.addupdate_scatter(accs[2], [isrc], es_d)
        plsc.addupdate_scatter(accs[3], [isrc], wv * t_d)
        if with_deg:
            plsc.addupdate_scatter(accs[4], [idst], ones16)

    nq = len(accs)
    outcopies = [
        pltpu.make_async_copy(
            a, out_hbm.at[pl.ds((wid * nq + q) * _NPAD, _NPAD)], sems.at[q])
        for q, a in enumerate(accs)
    ]
    for c in outcopies:
        c.start()
    for c in outcopies:
        c.wait()


@functools.lru_cache(maxsize=None)
def _make_edge(with_deg):
    nq = 5 if with_deg else 4
    scratch = [
        pltpu.VMEM((2 * _NPAD,), jnp.float32),
        pltpu.VMEM((_EPW,), jnp.int32),
        pltpu.VMEM((_EPW,), jnp.int32),
        pltpu.VMEM((_EPW,), jnp.float32),
        pltpu.SemaphoreType.DMA((5,)),
    ] + [pltpu.VMEM((_NPAD,), jnp.float32) for _ in range(nq)]
    return pl.kernel(
        functools.partial(_edge_body, with_deg),
        out_type=jax.ShapeDtypeStruct((_NW * nq * _NPAD,), jnp.float32),
        mesh=plsc.VectorSubcoreMesh(core_axis_name="c", subcore_axis_name="s",
                                    num_cores=_NC, num_subcores=_NS),
        scratch_types=scratch,
        compiler_params=pltpu.CompilerParams(needs_layout_passes=False),
    )


def _edge_deg(*args):
    return _make_edge(True)(*args)


def _edge_nodeg(*args):
    return _make_edge(False)(*args)


def _node_body(alpha, final, parts_ref, s3_ref, orig3_ref,
               U1_ref, ub1_ref, ug_ref, ube_ref, U2_ref, ub2_ref, U3_ref,
               ub3_ref, *rest):
    nq = 4 if final else 5
    if final:
        scal_ref, deg_ref, out_ref, acc_ref = rest
    else:
        out_ref, deg_out_ref, acc_ref = rest

    g = pl.program_id(0)
    chunk = parts_ref[0:nq * _NPAD]
    for k in range(1, _GB):
        chunk = chunk + parts_ref[k * nq * _NPAD:(k + 1) * nq * _NPAD]

    @pl.when(g == 0)
    def _():
        acc_ref[...] = chunk

    @pl.when(g > 0)
    def _():
        acc_ref[...] = acc_ref[...] + chunk

    @pl.when(g == _GN - 1)
    def _():
        acc = acc_ref[...]

        def plane(q):
            return acc[q * _NPAD:(q + 1) * _NPAD].reshape(1, _NPAD)

        den_in = plane(0)
        num_in = plane(1)
        den_out = plane(2)
        num_out = plane(3)
        s_in = num_in / jnp.maximum(den_in, 1e-20)
        s_out = num_out / jnp.maximum(den_out, 1e-20)
        s = s3_ref[:, 0:_NPAD]

        mask = (lax.broadcasted_iota(jnp.int32, (1, _NPAD), 1) < _N).astype(
            jnp.float32)

        U1T = U1_ref[...].T
        ub1 = ub1_ref[...].reshape(-1, 1)
        ug = ug_ref[...].reshape(-1, 1)
        ube = ube_ref[...].reshape(-1, 1)
        U2T = U2_ref[...].T
        ub2 = ub2_ref[...].reshape(-1, 1)
        U3T = U3_ref[...].T
        ub3 = ub3_ref[...].reshape(1, 1)

        u1 = (U1T[:, 0:1] * s + U1T[:, 1:2] * s_in
              + U1T[:, 2:3] * s_out + ub1)
        u1 = u1 * mask
        m = jnp.sum(u1, axis=1, keepdims=True) * (1.0 / _N)
        v = jnp.sum(u1 * u1, axis=1, keepdims=True) * (1.0 / _N) - m * m
        h = _leaky((u1 - m) / jnp.sqrt(v + 1e-5) * ug + ube, 0.1)
        u2 = _leaky(
            jnp.dot(U2T, h, preferred_element_type=jnp.float32) + ub2, 0.1)
        u3 = jax.nn.sigmoid(
            jnp.dot(U3T, u2, preferred_element_type=jnp.float32) + ub3)
        orig = orig3_ref[:, 0:_NPAD]
        sn = alpha * u3 + (1.0 - alpha) * orig

        if not final:
            es = jnp.exp(sn)
            out_ref[:, 0:_NPAD] = sn
            out_ref[:, _NPAD:2 * _NPAD] = es
            out_ref[:, 2 * _NPAD:3 * _NPAD] = es * sn
            deg_out_ref[...] = plane(4)
        else:
            scal = scal_ref[...]
            gamma_p = scal[0, 0]
            beta_p = scal[0, 1]
            cscale = scal[0, 2]
            mixraw = scal[0, 3]
            deg = deg_ref[...]
            c_v = jnp.log(deg * cscale + 1e-6)
            c_smooth = 5.0 * jnp.tanh((gamma_p * c_v + beta_p) * 0.2)
            mix = jax.nn.sigmoid(mixraw)
            fin = (mix * (sn * jax.nn.sigmoid(c_smooth))
                   + (1.0 - mix) * sn)
            out_ref[...] = fin[:, 0:_N]


def _full2d(shape):
    return pl.BlockSpec(shape, lambda w: (0, 0))


def _node_first(parts, s3, orig3, U1l, ub1l, ugl, ubel, U2l, ub2l, U3l,
                ub3l):
    nq = 5
    return pl.pallas_call(
        functools.partial(_node_body, 0.8, False),
        grid=(_GN,),
        in_specs=[
            pl.BlockSpec((_GB * nq * _NPAD,), lambda w: (w,)),
            _full2d((1, 3 * _NPAD)),
            _full2d((1, 3 * _NPAD)),
            pl.BlockSpec((3, 24), lambda w: (0, 0)),
            pl.BlockSpec((24,), lambda w: (0,)),
            pl.BlockSpec((24,), lambda w: (0,)),
            pl.BlockSpec((24,), lambda w: (0,)),
            pl.BlockSpec((24, 12), lambda w: (0, 0)),
            pl.BlockSpec((12,), lambda w: (0,)),
            pl.BlockSpec((12, 1), lambda w: (0, 0)),
            pl.BlockSpec((1,), lambda w: (0,)),
        ],
        out_specs=(_full2d((1, 3 * _NPAD)), _full2d((1, _NPAD))),
        out_shape=(
            jax.ShapeDtypeStruct((1, 3 * _NPAD), jnp.float32),
            jax.ShapeDtypeStruct((1, _NPAD), jnp.float32),
        ),
        scratch_shapes=[pltpu.VMEM((nq * _NPAD,), jnp.float32)],
    )(parts, s3, orig3, U1l, ub1l, ugl, ubel, U2l, ub2l, U3l, ub3l)


def _node_final(parts, s3, orig3, U1l, ub1l, ugl, ubel, U2l, ub2l, U3l,
                ub3l, scal, deg_row):
    nq = 4
    return pl.pallas_call(
        functools.partial(_node_body, 0.9, True),
        grid=(_GN,),
        in_specs=[
            pl.BlockSpec((_GB * nq * _NPAD,), lambda w: (w,)),
            _full2d((1, 3 * _NPAD)),
            _full2d((1, 3 * _NPAD)),
            pl.BlockSpec((3, 24), lambda w: (0, 0)),
            pl.BlockSpec((24,), lambda w: (0,)),
            pl.BlockSpec((24,), lambda w: (0,)),
            pl.BlockSpec((24,), lambda w: (0,)),
            pl.BlockSpec((24, 12), lambda w: (0, 0)),
            pl.BlockSpec((12,), lambda w: (0,)),
            pl.BlockSpec((12, 1), lambda w: (0, 0)),
            pl.BlockSpec((1,), lambda w: (0,)),
            _full2d((1, 4)),
            _full2d((1, _NPAD)),
        ],
        out_specs=_full2d((1, _N)),
        out_shape=jax.ShapeDtypeStruct((1, _N), jnp.float32),
        scratch_shapes=[pltpu.VMEM((nq * _NPAD,), jnp.float32)],
    )(parts, s3, orig3, U1l, ub1l, ugl, ubel, U2l, ub2l, U3l, ub3l,
      scal, deg_row)


def kernel(embeddings, edge_index, edge_weight, W1, b1, g1, be1, W2, b2, g2,
           be2, W3, b3, U1, ub1, ug, ube, U2, ub2, U3, ub3, gamma_p, beta_p,
           cscale, mixraw):
    src = edge_index[0]
    dst = edge_index[1]

    s3 = _scoring(embeddings, W1, b1, g1, be1, W2, b2, g2, be2, W3, b3)

    scal = jnp.stack([gamma_p, beta_p, cscale, mixraw]).reshape(1, 4)

    def layer_args(l):
        return (U1[l], ub1[l], ug[l], ube[l], U2[l], ub2[l], U3[l], ub3[l])

    parts0 = _edge_deg(s3.reshape(3 * _NPAD), src, dst, edge_weight)
    s3n, deg_row = _node_first(parts0, s3, s3, *layer_args(0))

    parts1 = _edge_nodeg(s3n.reshape(3 * _NPAD), src, dst, edge_weight)
    fin = _node_final(parts1, s3n, s3, *layer_args(1), scal, deg_row)

    return fin.reshape(_N)
